# Initial kernel scaffold; baseline (speedup 1.0000x reference)
import jax
import jax.numpy as jnp
from jax.experimental import pallas as pl


def kernel(x, hyperedge_index, weight, att):
    raise NotImplementedError("write your pallas kernel here")


# speedup vs baseline: 14.8516x; 14.8516x over previous
import jax
import jax.numpy as jnp
from jax import lax
from jax.experimental import pallas as pl
from jax.experimental.pallas import tpu as pltpu
from jax.experimental.pallas import tpu_sc as plsc

H = 8
DH = 64
C = 512
NCH = 4
CW = 128
N = 10000
E = 160000
B = 4
NEG = 0.2

NC = 2
NS = 16
NW = NC * NS
L = 16

EPAD = 163840
EPW = EPAD // NW
TB = 128
NBATCH = EPW // TB
RPT = 625
FL = 125

BN = 400
NT = N // BN

_MESH = plsc.VectorSubcoreMesh(core_axis_name="c", subcore_axis_name="s")
_SCPAR = pltpu.CompilerParams(use_tc_tiling_on_sc=False)

_GDN = lax.GatherDimensionNumbers(
    offset_dims=(), collapsed_slice_dims=(0,), start_index_map=(0,))


def _vgather(v, idx):
  return lax.gather(v, idx[:, None], _GDN, (1,),
                    mode=lax.GatherScatterMode.PROMISE_IN_BOUNDS)


def _proj_body(xr, wr, pr, xcr, abr):
  c = pl.program_id(2)
  m = jnp.dot(xr[0], wr[...], preferred_element_type=jnp.float32)
  xcr[0] = m

  @pl.when(c == 0)
  def _():
    abr[0] = jnp.zeros((BN, 2 * H), jnp.float32)

  abr[0] += jnp.dot(m, pr[...], preferred_element_type=jnp.float32)


def _project(x, weight, p):
  return pl.pallas_call(
      _proj_body,
      grid=(B, NT, NCH),
      in_specs=[
          pl.BlockSpec((1, BN, C), lambda b, n, c: (b, n, 0)),
          pl.BlockSpec((C, CW), lambda b, n, c: (0, c)),
          pl.BlockSpec((CW, 2 * H), lambda b, n, c: (c, 0)),
      ],
      out_specs=[
          pl.BlockSpec((1, BN, CW), lambda b, n, c: (b, c * NT + n, 0)),
          pl.BlockSpec((1, BN, 2 * H), lambda b, n, c: (b, n, 0)),
      ],
      out_shape=[
          jax.ShapeDtypeStruct((B, NCH * N, CW), jnp.float32),
          jax.ShapeDtypeStruct((B, N, 2 * H), jnp.float32),
      ],
  )(x, weight, p)


def _edge_stats_body(src_hbm, dst_hbm, ab_hbm, ew_hbm, accp_hbm,
                     acc, sbuf, dbuf, sgb, dgb, sarow, darow, rows, drows,
                     zbuf, gs0, gd0, gs1, gd1, ss0, sd0, ss1, sd1):
  core = lax.axis_index("c")
  sid = lax.axis_index("s")
  wid = sid * NC + core
  off = wid * EPW
  gsem = ((gs0, gd0), (gs1, gd1))
  ssem = ((ss0, sd0), (ss1, sd1))

  @pl.loop(0, RPT)
  def _(r):
    zbuf[r] = jnp.zeros((L,), jnp.float32)

  for b in range(B):
    pltpu.sync_copy(zbuf, acc.at[pl.ds(sid * RPT, RPT)])
    plsc.subcore_barrier()

    def load_idx(bi, p, b=b):
      gl = off + bi * TB
      pltpu.sync_copy(src_hbm.at[pl.ds(b * EPAD + gl, TB)], sbuf.at[p])
      pltpu.sync_copy(dst_hbm.at[pl.ds(b * EPAD + gl, TB)], dbuf.at[p])

      @pl.loop(0, TB // L)
      def _(v):
        sgb[p, pl.ds(v * L, L)] = sbuf[p, pl.ds(v * L, L)] + (b * N)
        dgb[p, pl.ds(v * L, L)] = dbuf[p, pl.ds(v * L, L)] + (b * N)

    def gathers(p):
      c1 = pltpu.async_copy(ab_hbm.at[sgb.at[p]], sarow.at[p], gsem[p][0])
      c2 = pltpu.async_copy(ab_hbm.at[dgb.at[p]], darow.at[p], gsem[p][1])
      return c1, c2

    def rowloop(gl, p):
      @pl.loop(0, TB)
      def _(j):
        iota = lax.iota(jnp.int32, L)
        perm = jnp.bitwise_and(iota + 8, 15)
        one8 = jnp.where(iota == 8, 1.0, 0.0).astype(jnp.float32)
        drow_c = jnp.where(iota == 9, 1.0, 0.0).astype(jnp.float32)
        va = sarow[p, j]
        vb = darow[p, j]
        s = va + _vgather(vb, perm)
        s = jnp.where(s > 0, s, NEG * s)
        ex = jnp.exp(s)
        r = jnp.where(iota < 8, ex, one8)
        vm = jnp.full((L,), jnp.where(gl + j < E, 1.0, 0.0), jnp.float32)
        rows[p, j] = r * vm
        drows[p, j] = drow_c * vm

    def scatters(p):
      c1 = pltpu.async_copy(rows.at[p], acc.at[sbuf.at[p]], ssem[p][0],
                            add=True)
      c2 = pltpu.async_copy(drows.at[p], acc.at[dbuf.at[p]], ssem[p][1],
                            add=True)
      return c1, c2

    @pl.loop(0, NBATCH // 2)
    def _(t, b=b):
      bi0 = t * 2
      gl0 = off + bi0 * TB
      gl1 = gl0 + TB
      load_idx(bi0, 0)
      g0 = gathers(0)
      load_idx(bi0 + 1, 1)
      g1 = gathers(1)
      g0[0].wait()
      g0[1].wait()
      rowloop(gl0, 0)
      s0 = scatters(0)
      pltpu.sync_copy(rows.at[0], ew_hbm.at[pl.ds(b * EPAD + gl0, TB)])
      g1[0].wait()
      g1[1].wait()
      rowloop(gl1, 1)
      s1 = scatters(1)
      pltpu.sync_copy(rows.at[1], ew_hbm.at[pl.ds(b * EPAD + gl1, TB)])
      s0[0].wait()
      s0[1].wait()
      s1[0].wait()
      s1[1].wait()

    plsc.subcore_barrier()
    r0 = sid * RPT
    pltpu.sync_copy(acc.at[pl.ds(r0, RPT)],
                    accp_hbm.at[pl.ds((b * NC + core) * N + r0, RPT)])
    plsc.subcore_barrier()


_edge_stats = pl.kernel(
    _edge_stats_body,
    out_type=[
        jax.ShapeDtypeStruct((B * EPAD, L), jnp.float32),
        jax.ShapeDtypeStruct((B * NC * N, L), jnp.float32),
    ],
    mesh=_MESH,
    scratch_types=[
        pltpu.VMEM_SHARED((N, L), jnp.float32),
        pltpu.VMEM((2, TB), jnp.int32),
        pltpu.VMEM((2, TB), jnp.int32),
        pltpu.VMEM((2, TB), jnp.int32),
        pltpu.VMEM((2, TB), jnp.int32),
        pltpu.VMEM((2, TB, L), jnp.float32),
        pltpu.VMEM((2, TB, L), jnp.float32),
        pltpu.VMEM((2, TB, L), jnp.float32),
        pltpu.VMEM((2, TB, L), jnp.float32),
        pltpu.VMEM((RPT, L), jnp.float32),
        pltpu.SemaphoreType.DMA,
        pltpu.SemaphoreType.DMA,
        pltpu.SemaphoreType.DMA,
        pltpu.SemaphoreType.DMA,
        pltpu.SemaphoreType.DMA,
        pltpu.SemaphoreType.DMA,
        pltpu.SemaphoreType.DMA,
        pltpu.SemaphoreType.DMA,
    ],
    compiler_params=_SCPAR,
)


NPW = 312


def _scales_body(accp_hbm, sc_hbm, b0, b1, b2):
  core = lax.axis_index("c")
  sid = lax.axis_index("s")
  wid = sid * NC + core

  for b in range(B):

    def run(base, nrows, b=b):
      pltpu.sync_copy(accp_hbm.at[pl.ds(b * NC * N + base, nrows)],
                      b0.at[pl.ds(0, nrows)])
      pltpu.sync_copy(accp_hbm.at[pl.ds((b * NC + 1) * N + base, nrows)],
                      b1.at[pl.ds(0, nrows)])

      @pl.loop(0, nrows)
      def _(r):
        iota = lax.iota(jnp.int32, L)
        p = b0[r] + b1[r]
        pp = jnp.maximum(p, 1e-18)
        inv = p / (pp * pp)
        out = jnp.where(iota == 8, p, inv)
        b2[r] = jnp.where(iota < 10, out, 0.0)

      pltpu.sync_copy(b2.at[pl.ds(0, nrows)],
                      sc_hbm.at[pl.ds(b * N + base, nrows)])

    run(wid * NPW, NPW)

    @pl.when(wid == NW - 1)
    def _():
      run(NW * NPW, N - NW * NPW)


_scales = pl.kernel(
    _scales_body,
    out_type=[jax.ShapeDtypeStruct((B * N, L), jnp.float32)],
    mesh=_MESH,
    scratch_types=[
        pltpu.VMEM((NPW, L), jnp.float32),
        pltpu.VMEM((NPW, L), jnp.float32),
        pltpu.VMEM((NPW, L), jnp.float32),
    ],
    compiler_params=_SCPAR,
)


def _ahat_body(src_hbm, ew_hbm, sc_hbm, ah_hbm, sbuf, sgb, erow, srow,
               g0, g1):
  core = lax.axis_index("c")
  sid = lax.axis_index("s")
  off = (sid * NC + core) * EPW
  gsem = (g0, g1)

  for b in range(B):

    def load(bi, p, b=b):
      gl = off + bi * TB
      pltpu.sync_copy(src_hbm.at[pl.ds(b * EPAD + gl, TB)], sbuf.at[p])
      pltpu.sync_copy(ew_hbm.at[pl.ds(b * EPAD + gl, TB)], erow.at[p])

      @pl.loop(0, TB // L)
      def _(v):
        sgb[p, pl.ds(v * L, L)] = sbuf[p, pl.ds(v * L, L)] + (b * N)

      return pltpu.async_copy(sc_hbm.at[sgb.at[p]], srow.at[p], gsem[p])

    def mulout(bi, p, b=b):
      @pl.loop(0, TB)
      def _(j):
        erow[p, j] = erow[p, j] * srow[p, j]

      gl = off + bi * TB
      pltpu.sync_copy(erow.at[p], ah_hbm.at[pl.ds(b * EPAD + gl, TB)])

    @pl.loop(0, NBATCH // 2)
    def _(t):
      bi0 = t * 2
      d0 = load(bi0, 0)
      d1 = load(bi0 + 1, 1)
      d0.wait()
      mulout(bi0, 0)
      d1.wait()
      mulout(bi0 + 1, 1)


_ahat = pl.kernel(
    _ahat_body,
    out_type=[jax.ShapeDtypeStruct((B * EPAD, L), jnp.float32)],
    mesh=_MESH,
    scratch_types=[
        pltpu.VMEM((2, TB), jnp.int32),
        pltpu.VMEM((2, TB), jnp.int32),
        pltpu.VMEM((2, TB, L), jnp.float32),
        pltpu.VMEM((2, TB, L), jnp.float32),
        pltpu.SemaphoreType.DMA,
        pltpu.SemaphoreType.DMA,
    ],
    compiler_params=_SCPAR,
)


def _prop_body(gi_hbm, si_hbm, ah_hbm, tab_hbm, qp_hbm,
               acc, gbuf, gbufc, sbuf, arow, feat,
               gs0, gs1, ss0, ss1):
  core = lax.axis_index("c")
  sid = lax.axis_index("s")
  off = (sid * NC + core) * EPW
  gsem = (gs0, gs1)
  ssem = (ss0, ss1)

  for b in range(B):
    for c in range(NCH):
      toff = (b * NCH + c) * N

      @pl.loop(0, FL)
      def _(r):
        for k in range(CW // L):
          feat[0, r, pl.ds(k * L, L)] = jnp.zeros((L,), jnp.float32)

      for f in range(RPT // FL):
        pltpu.sync_copy(feat.at[0, pl.ds(0, FL)],
                        acc.at[pl.ds(sid * RPT + f * FL, FL)])
      plsc.subcore_barrier()

      def load_idx(bi, p, b=b, toff=toff):
        gl = off + bi * TB
        pltpu.sync_copy(gi_hbm.at[pl.ds(b * EPAD + gl, TB)], gbuf.at[p])
        pltpu.sync_copy(si_hbm.at[pl.ds(b * EPAD + gl, TB)], sbuf.at[p])
        pltpu.sync_copy(ah_hbm.at[pl.ds(b * EPAD + gl, TB)], arow.at[p])

        @pl.loop(0, TB // L)
        def _(v):
          gbufc[p, pl.ds(v * L, L)] = gbuf[p, pl.ds(v * L, L)] + toff

      def rowloop(p, c=c):
        @pl.loop(0, TB)
        def _(j):
          a = arow[p, j]
          w0 = _vgather(a, jnp.full((L,), 2 * c, jnp.int32))
          w1 = _vgather(a, jnp.full((L,), 2 * c + 1, jnp.int32))
          for k in range(CW // L):
            w = w0 if k < (CW // L // 2) else w1
            feat[p, j, pl.ds(k * L, L)] = feat[p, j, pl.ds(k * L, L)] * w

      @pl.loop(0, NBATCH // 2)
      def _(t):
        bi0 = t * 2
        load_idx(bi0, 0)
        d0 = pltpu.async_copy(tab_hbm.at[gbufc.at[0]], feat.at[0], gsem[0])
        load_idx(bi0 + 1, 1)
        d1 = pltpu.async_copy(tab_hbm.at[gbufc.at[1]], feat.at[1], gsem[1])
        d0.wait()
        rowloop(0)
        s0 = pltpu.async_copy(feat.at[0], acc.at[sbuf.at[0]], ssem[0],
                              add=True)
        d1.wait()
        rowloop(1)
        s1 = pltpu.async_copy(feat.at[1], acc.at[sbuf.at[1]], ssem[1],
                              add=True)
        s0.wait()
        s1.wait()

      plsc.subcore_barrier()
      for f in range(RPT // FL):
        r0 = sid * RPT + f * FL
        pltpu.sync_copy(
            acc.at[pl.ds(r0, FL)],
            qp_hbm.at[pl.ds(((b * NC + core) * NCH + c) * N + r0, FL)])
      plsc.subcore_barrier()


_propagate = pl.kernel(
    _prop_body,
    out_type=[jax.ShapeDtypeStruct((B * NC * NCH * N, CW), jnp.float32)],
    mesh=_MESH,
    scratch_types=[
        pltpu.VMEM_SHARED((N, CW), jnp.float32),
        pltpu.VMEM((2, TB), jnp.int32),
        pltpu.VMEM((2, TB), jnp.int32),
        pltpu.VMEM((2, TB), jnp.int32),
        pltpu.VMEM((2, TB, L), jnp.float32),
        pltpu.VMEM((2, TB, CW), jnp.float32),
        pltpu.SemaphoreType.DMA,
        pltpu.SemaphoreType.DMA,
        pltpu.SemaphoreType.DMA,
        pltpu.SemaphoreType.DMA,
    ],
    compiler_params=_SCPAR,
)


def _combine_e_body(qr, scr, orr):
  s = qr[0, 0] + qr[0, 1]
  orr[0] = s * scr[0][:, 9:10]


def _combine_e(qp, scales):
  return pl.pallas_call(
      _combine_e_body,
      grid=(B, NCH, NT),
      in_specs=[
          pl.BlockSpec((1, NC, BN, CW), lambda b, c, n: (b, 0, c * NT + n, 0)),
          pl.BlockSpec((1, BN, L), lambda b, c, n: (b, n, 0)),
      ],
      out_specs=pl.BlockSpec((1, BN, CW), lambda b, c, n: (b, c * NT + n, 0)),
      out_shape=jax.ShapeDtypeStruct((B, NCH * N, CW), jnp.float32),
  )(qp, scales)


def _combine_n_body(qr, scr, orr):
  s = qr[0, 0] + qr[0, 1]
  orr[0] = s * scr[0][:, 8:9]


def _combine_n(rp, scales):
  return pl.pallas_call(
      _combine_n_body,
      grid=(B, NCH, NT),
      in_specs=[
          pl.BlockSpec((1, NC, BN, CW), lambda b, c, n: (b, 0, c * NT + n, 0)),
          pl.BlockSpec((1, BN, L), lambda b, c, n: (b, n, 0)),
      ],
      out_specs=pl.BlockSpec((1, BN, CW), lambda b, c, n: (b, n, c)),
      out_shape=jax.ShapeDtypeStruct((B, N, C), jnp.float32),
  )(rp, scales)


def kernel(x, hyperedge_index, weight, att):
  x = x.astype(jnp.float32)
  hi = hyperedge_index.astype(jnp.int32)
  weight = weight.astype(jnp.float32)
  att = att.astype(jnp.float32)

  att_a = att[0, :, :DH]
  att_b = att[0, :, DH:]
  eye = jnp.eye(H, dtype=jnp.float32)
  pa = (att_a[:, :, None] * eye[:, None, :]).reshape(C, H)
  pb = (att_b[:, :, None] * eye[:, None, :]).reshape(C, H)
  p = jnp.concatenate([pa, pb], axis=1)

  src = jnp.pad(hi[:, 0, :], ((0, 0), (0, EPAD - E))).reshape(B * EPAD)
  dst = jnp.pad(hi[:, 1, :], ((0, 0), (0, EPAD - E))).reshape(B * EPAD)

  xpc, ab = _project(x, weight, p)
  xpc_f = xpc.reshape(B * NCH * N, CW)
  ab_f = ab.reshape(B * N, 2 * H)

  ew, accp = _edge_stats(src, dst, ab_f)
  (scales,) = _scales(accp)
  (ahat,) = _ahat(src, ew, scales)
  (qp,) = _propagate(src, dst, ahat, xpc_f)
  oe = _combine_e(qp.reshape(B, NC * NCH * N, CW).reshape(B, NC, NCH * N, CW),
                  scales.reshape(B, N, L))
  (rp,) = _propagate(dst, src, ahat, oe.reshape(B * NCH * N, CW))
  out = _combine_n(rp.reshape(B, NC, NCH * N, CW), scales.reshape(B, N, L))
  return out


# speedup vs baseline: 16.0617x; 1.0815x over previous
import jax
import jax.numpy as jnp
from jax import lax
from jax.experimental import pallas as pl
from jax.experimental.pallas import tpu as pltpu
from jax.experimental.pallas import tpu_sc as plsc

H = 8
DH = 64
C = 512
NCH = 4
CW = 128
N = 10000
E = 160000
B = 4
NEG = 0.2

NC = 2
NS = 16
NW = NC * NS
L = 16

EPAD = 163840
EPW = EPAD // NW
TB = 128
NBATCH = EPW // TB
NB0 = 52
NB1 = 28
RPT = 625
FL = 125

BN = 400
NT = N // BN

_MESH = plsc.VectorSubcoreMesh(core_axis_name="c", subcore_axis_name="s")
_SCPAR = pltpu.CompilerParams(use_tc_tiling_on_sc=False)

_GDN = lax.GatherDimensionNumbers(
    offset_dims=(), collapsed_slice_dims=(0,), start_index_map=(0,))


def _vgather(v, idx):
  return lax.gather(v, idx[:, None], _GDN, (1,),
                    mode=lax.GatherScatterMode.PROMISE_IN_BOUNDS)


def _proj_body(xr, wr, pr, xcr, abr):
  c = pl.program_id(2)
  m = jnp.dot(xr[0], wr[...], preferred_element_type=jnp.float32)
  xcr[0] = m

  @pl.when(c == 0)
  def _():
    abr[0] = jnp.zeros((BN, 2 * H), jnp.float32)

  abr[0] += jnp.dot(m, pr[...], preferred_element_type=jnp.float32)


def _project(x, weight, p):
  return pl.pallas_call(
      _proj_body,
      grid=(B, NT, NCH),
      in_specs=[
          pl.BlockSpec((1, BN, C), lambda b, n, c: (b, n, 0)),
          pl.BlockSpec((C, CW), lambda b, n, c: (0, c)),
          pl.BlockSpec((CW, 2 * H), lambda b, n, c: (c, 0)),
      ],
      out_specs=[
          pl.BlockSpec((1, BN, CW), lambda b, n, c: (b, c * NT + n, 0)),
          pl.BlockSpec((1, BN, 2 * H), lambda b, n, c: (b, n, 0)),
      ],
      out_shape=[
          jax.ShapeDtypeStruct((B, NCH * N, CW), jnp.float32),
          jax.ShapeDtypeStruct((B, N, 2 * H), jnp.float32),
      ],
  )(x, weight, p)


def _edge_stats_body(src_hbm, dst_hbm, ab_hbm, ew_hbm, accp_hbm,
                     acc, sbuf, dbuf, sgb, dgb, sarow, darow, rows, drows,
                     zbuf, gs0, gd0, gs1, gd1, ss0, sd0, ss1, sd1):
  core = lax.axis_index("c")
  sid = lax.axis_index("s")
  off = jnp.where(core == 0, sid * NB0, NS * NB0 + sid * NB1) * TB
  nbp = jnp.where(core == 0, NB0 // 2, NB1 // 2)
  gsem = ((gs0, gd0), (gs1, gd1))
  ssem = ((ss0, sd0), (ss1, sd1))

  @pl.loop(0, RPT)
  def _(r):
    zbuf[r] = jnp.zeros((L,), jnp.float32)

  for b in range(B):
    pltpu.sync_copy(zbuf, acc.at[pl.ds(sid * RPT, RPT)])
    plsc.subcore_barrier()

    def load_idx(bi, p, b=b):
      gl = off + bi * TB
      pltpu.sync_copy(src_hbm.at[pl.ds(b * EPAD + gl, TB)], sbuf.at[p])
      pltpu.sync_copy(dst_hbm.at[pl.ds(b * EPAD + gl, TB)], dbuf.at[p])

      @pl.loop(0, TB // L)
      def _(v):
        sgb[p, pl.ds(v * L, L)] = sbuf[p, pl.ds(v * L, L)] + (b * N)
        dgb[p, pl.ds(v * L, L)] = dbuf[p, pl.ds(v * L, L)] + (b * N)

    def gathers(p):
      c1 = pltpu.async_copy(ab_hbm.at[sgb.at[p]], sarow.at[p], gsem[p][0])
      c2 = pltpu.async_copy(ab_hbm.at[dgb.at[p]], darow.at[p], gsem[p][1])
      return c1, c2

    def rowloop(gl, p):
      @pl.loop(0, TB)
      def _(j):
        iota = lax.iota(jnp.int32, L)
        perm = jnp.bitwise_and(iota + 8, 15)
        one8 = jnp.where(iota == 8, 1.0, 0.0).astype(jnp.float32)
        drow_c = jnp.where(iota == 9, 1.0, 0.0).astype(jnp.float32)
        va = sarow[p, j]
        vb = darow[p, j]
        s = va + _vgather(vb, perm)
        s = jnp.where(s > 0, s, NEG * s)
        ex = jnp.exp(s)
        r = jnp.where(iota < 8, ex, one8)
        vm = jnp.full((L,), jnp.where(gl + j < E, 1.0, 0.0), jnp.float32)
        rows[p, j] = r * vm
        drows[p, j] = drow_c * vm

    def scatters(p):
      c1 = pltpu.async_copy(rows.at[p], acc.at[sbuf.at[p]], ssem[p][0],
                            add=True)
      c2 = pltpu.async_copy(drows.at[p], acc.at[dbuf.at[p]], ssem[p][1],
                            add=True)
      return c1, c2

    @pl.loop(0, nbp)
    def _(t, b=b):
      bi0 = t * 2
      gl0 = off + bi0 * TB
      gl1 = gl0 + TB
      load_idx(bi0, 0)
      g0 = gathers(0)
      load_idx(bi0 + 1, 1)
      g1 = gathers(1)
      g0[0].wait()
      g0[1].wait()
      rowloop(gl0, 0)
      s0 = scatters(0)
      pltpu.sync_copy(rows.at[0], ew_hbm.at[pl.ds(b * EPAD + gl0, TB)])
      g1[0].wait()
      g1[1].wait()
      rowloop(gl1, 1)
      s1 = scatters(1)
      pltpu.sync_copy(rows.at[1], ew_hbm.at[pl.ds(b * EPAD + gl1, TB)])
      s0[0].wait()
      s0[1].wait()
      s1[0].wait()
      s1[1].wait()

    plsc.subcore_barrier()
    r0 = sid * RPT
    pltpu.sync_copy(acc.at[pl.ds(r0, RPT)],
                    accp_hbm.at[pl.ds((b * NC + core) * N + r0, RPT)])
    plsc.subcore_barrier()


_edge_stats = pl.kernel(
    _edge_stats_body,
    out_type=[
        jax.ShapeDtypeStruct((B * EPAD, L), jnp.float32),
        jax.ShapeDtypeStruct((B * NC * N, L), jnp.float32),
    ],
    mesh=_MESH,
    scratch_types=[
        pltpu.VMEM_SHARED((N, L), jnp.float32),
        pltpu.VMEM((2, TB), jnp.int32),
        pltpu.VMEM((2, TB), jnp.int32),
        pltpu.VMEM((2, TB), jnp.int32),
        pltpu.VMEM((2, TB), jnp.int32),
        pltpu.VMEM((2, TB, L), jnp.float32),
        pltpu.VMEM((2, TB, L), jnp.float32),
        pltpu.VMEM((2, TB, L), jnp.float32),
        pltpu.VMEM((2, TB, L), jnp.float32),
        pltpu.VMEM((RPT, L), jnp.float32),
        pltpu.SemaphoreType.DMA,
        pltpu.SemaphoreType.DMA,
        pltpu.SemaphoreType.DMA,
        pltpu.SemaphoreType.DMA,
        pltpu.SemaphoreType.DMA,
        pltpu.SemaphoreType.DMA,
        pltpu.SemaphoreType.DMA,
        pltpu.SemaphoreType.DMA,
    ],
    compiler_params=_SCPAR,
)


NPW = 312


def _scales_body(accp_hbm, sc_hbm, b0, b1, b2):
  core = lax.axis_index("c")
  sid = lax.axis_index("s")
  wid = sid * NC + core

  for b in range(B):

    def run(base, nrows, b=b):
      pltpu.sync_copy(accp_hbm.at[pl.ds(b * NC * N + base, nrows)],
                      b0.at[pl.ds(0, nrows)])
      pltpu.sync_copy(accp_hbm.at[pl.ds((b * NC + 1) * N + base, nrows)],
                      b1.at[pl.ds(0, nrows)])

      @pl.loop(0, nrows)
      def _(r):
        iota = lax.iota(jnp.int32, L)
        p = b0[r] + b1[r]
        pp = jnp.maximum(p, 1e-18)
        inv = p / (pp * pp)
        out = jnp.where(iota == 8, p, inv)
        b2[r] = jnp.where(iota < 10, out, 0.0)

      pltpu.sync_copy(b2.at[pl.ds(0, nrows)],
                      sc_hbm.at[pl.ds(b * N + base, nrows)])

    run(wid * NPW, NPW)

    @pl.when(wid == NW - 1)
    def _():
      run(NW * NPW, N - NW * NPW)


_scales = pl.kernel(
    _scales_body,
    out_type=[jax.ShapeDtypeStruct((B * N, L), jnp.float32)],
    mesh=_MESH,
    scratch_types=[
        pltpu.VMEM((NPW, L), jnp.float32),
        pltpu.VMEM((NPW, L), jnp.float32),
        pltpu.VMEM((NPW, L), jnp.float32),
    ],
    compiler_params=_SCPAR,
)


def _ahat_body(src_hbm, ew_hbm, sc_hbm, ah_hbm, sbuf, sgb, erow, srow,
               g0, g1):
  core = lax.axis_index("c")
  sid = lax.axis_index("s")
  off = jnp.where(core == 0, sid * NB0, NS * NB0 + sid * NB1) * TB
  nbp = jnp.where(core == 0, NB0 // 2, NB1 // 2)
  gsem = (g0, g1)

  for b in range(B):

    def load(bi, p, b=b):
      gl = off + bi * TB
      pltpu.sync_copy(src_hbm.at[pl.ds(b * EPAD + gl, TB)], sbuf.at[p])
      pltpu.sync_copy(ew_hbm.at[pl.ds(b * EPAD + gl, TB)], erow.at[p])

      @pl.loop(0, TB // L)
      def _(v):
        sgb[p, pl.ds(v * L, L)] = sbuf[p, pl.ds(v * L, L)] + (b * N)

      return pltpu.async_copy(sc_hbm.at[sgb.at[p]], srow.at[p], gsem[p])

    def mulout(bi, p, b=b):
      @pl.loop(0, TB)
      def _(j):
        erow[p, j] = erow[p, j] * srow[p, j]

      gl = off + bi * TB
      pltpu.sync_copy(erow.at[p], ah_hbm.at[pl.ds(b * EPAD + gl, TB)])

    @pl.loop(0, nbp)
    def _(t):
      bi0 = t * 2
      d0 = load(bi0, 0)
      d1 = load(bi0 + 1, 1)
      d0.wait()
      mulout(bi0, 0)
      d1.wait()
      mulout(bi0 + 1, 1)


_ahat = pl.kernel(
    _ahat_body,
    out_type=[jax.ShapeDtypeStruct((B * EPAD, L), jnp.float32)],
    mesh=_MESH,
    scratch_types=[
        pltpu.VMEM((2, TB), jnp.int32),
        pltpu.VMEM((2, TB), jnp.int32),
        pltpu.VMEM((2, TB, L), jnp.float32),
        pltpu.VMEM((2, TB, L), jnp.float32),
        pltpu.SemaphoreType.DMA,
        pltpu.SemaphoreType.DMA,
    ],
    compiler_params=_SCPAR,
)


BG = 4
TOTB = B * EPAD // TB


def _prop_body(gi_hbm, si_hbm, ah_hbm, tab_hbm, qp_hbm,
               acc, ggi, gsi, gah, feat, gs0, gs1, ss0, ss1):
  core = lax.axis_index("c")
  sid = lax.axis_index("s")
  lrow0 = jnp.where(core == 0, sid * NB0, NS * NB0 + sid * NB1)
  ngrp = jnp.where(core == 0, NB0 // BG, NB1 // BG)
  gsem = (gs0, gs1)
  ssem = (ss0, ss1)

  for b in range(B):
    for c in range(NCH):
      toff = (b * NCH + c) * N
      row0 = b * (EPAD // TB) + lrow0

      @pl.loop(0, FL)
      def _(r):
        for k in range(CW // L):
          feat[0, r, pl.ds(k * L, L)] = jnp.zeros((L,), jnp.float32)

      for f in range(RPT // FL):
        pltpu.sync_copy(feat.at[0, pl.ds(0, FL)],
                        acc.at[pl.ds(sid * RPT + f * FL, FL)])
      plsc.subcore_barrier()

      def rowloop(p, gk, c=c):
        @pl.loop(0, TB, unroll=2)
        def _(j):
          a = gah[gk, j]
          w0 = _vgather(a, jnp.full((L,), 2 * c, jnp.int32))
          w1 = _vgather(a, jnp.full((L,), 2 * c + 1, jnp.int32))
          for k in range(CW // L):
            w = w0 if k < (CW // L // 2) else w1
            feat[p, j, pl.ds(k * L, L)] = feat[p, j, pl.ds(k * L, L)] * w

      @pl.loop(0, ngrp)
      def _(g, toff=toff, row0=row0):
        r = row0 + g * BG
        pltpu.sync_copy(gi_hbm.at[pl.ds(r, BG)], ggi)
        pltpu.sync_copy(si_hbm.at[pl.ds(r, BG)], gsi)
        pltpu.sync_copy(ah_hbm.at[pl.ds(r, BG)], gah)

        @pl.loop(0, BG * TB // L)
        def _(v):
          i0 = v * L
          k0 = i0 // TB
          j0 = i0 % TB
          ggi[k0, pl.ds(j0, L)] = ggi[k0, pl.ds(j0, L)] + toff

        for pair in range(BG // 2):
          k0, k1 = pair * 2, pair * 2 + 1
          d0 = pltpu.async_copy(tab_hbm.at[ggi.at[k0]], feat.at[0], gsem[0])
          d1 = pltpu.async_copy(tab_hbm.at[ggi.at[k1]], feat.at[1], gsem[1])
          d0.wait()
          rowloop(0, k0)
          s0 = pltpu.async_copy(feat.at[0], acc.at[gsi.at[k0]], ssem[0],
                                add=True)
          d1.wait()
          rowloop(1, k1)
          s1 = pltpu.async_copy(feat.at[1], acc.at[gsi.at[k1]], ssem[1],
                                add=True)
          s0.wait()
          s1.wait()

      plsc.subcore_barrier()
      for f in range(RPT // FL):
        r0 = sid * RPT + f * FL
        pltpu.sync_copy(
            acc.at[pl.ds(r0, FL)],
            qp_hbm.at[pl.ds(((b * NC + core) * NCH + c) * N + r0, FL)])
      plsc.subcore_barrier()


_propagate = pl.kernel(
    _prop_body,
    out_type=[jax.ShapeDtypeStruct((B * NC * NCH * N, CW), jnp.float32)],
    mesh=_MESH,
    scratch_types=[
        pltpu.VMEM_SHARED((N, CW), jnp.float32),
        pltpu.VMEM((BG, TB), jnp.int32),
        pltpu.VMEM((BG, TB), jnp.int32),
        pltpu.VMEM((BG, TB, L), jnp.float32),
        pltpu.VMEM((2, TB, CW), jnp.float32),
        pltpu.SemaphoreType.DMA,
        pltpu.SemaphoreType.DMA,
        pltpu.SemaphoreType.DMA,
        pltpu.SemaphoreType.DMA,
    ],
    compiler_params=_SCPAR,
)


def _combine_e_body(qr, scr, orr):
  s = qr[0, 0] + qr[0, 1]
  orr[0] = s * scr[0][:, 9:10]


def _combine_e(qp, scales):
  return pl.pallas_call(
      _combine_e_body,
      grid=(B, NCH, NT),
      in_specs=[
          pl.BlockSpec((1, NC, BN, CW), lambda b, c, n: (b, 0, c * NT + n, 0)),
          pl.BlockSpec((1, BN, L), lambda b, c, n: (b, n, 0)),
      ],
      out_specs=pl.BlockSpec((1, BN, CW), lambda b, c, n: (b, c * NT + n, 0)),
      out_shape=jax.ShapeDtypeStruct((B, NCH * N, CW), jnp.float32),
  )(qp, scales)


def _combine_n_body(qr, scr, orr):
  s = qr[0, 0] + qr[0, 1]
  orr[0] = s * scr[0][:, 8:9]


def _combine_n(rp, scales):
  return pl.pallas_call(
      _combine_n_body,
      grid=(B, NCH, NT),
      in_specs=[
          pl.BlockSpec((1, NC, BN, CW), lambda b, c, n: (b, 0, c * NT + n, 0)),
          pl.BlockSpec((1, BN, L), lambda b, c, n: (b, n, 0)),
      ],
      out_specs=pl.BlockSpec((1, BN, CW), lambda b, c, n: (b, n, c)),
      out_shape=jax.ShapeDtypeStruct((B, N, C), jnp.float32),
  )(rp, scales)


def kernel(x, hyperedge_index, weight, att):
  x = x.astype(jnp.float32)
  hi = hyperedge_index.astype(jnp.int32)
  weight = weight.astype(jnp.float32)
  att = att.astype(jnp.float32)

  att_a = att[0, :, :DH]
  att_b = att[0, :, DH:]
  eye = jnp.eye(H, dtype=jnp.float32)
  pa = (att_a[:, :, None] * eye[:, None, :]).reshape(C, H)
  pb = (att_b[:, :, None] * eye[:, None, :]).reshape(C, H)
  p = jnp.concatenate([pa, pb], axis=1)

  src = jnp.pad(hi[:, 0, :], ((0, 0), (0, EPAD - E))).reshape(B * EPAD)
  dst = jnp.pad(hi[:, 1, :], ((0, 0), (0, EPAD - E))).reshape(B * EPAD)

  xpc, ab = _project(x, weight, p)
  xpc_f = xpc.reshape(B * NCH * N, CW)
  ab_f = ab.reshape(B * N, 2 * H)

  ew, accp = _edge_stats(src, dst, ab_f)
  (scales,) = _scales(accp)
  (ahat,) = _ahat(src, ew, scales)
  src2 = src.reshape(TOTB, TB)
  dst2 = dst.reshape(TOTB, TB)
  ah3 = ahat.reshape(TOTB, TB, L)
  (qp,) = _propagate(src2, dst2, ah3, xpc_f)
  oe = _combine_e(qp.reshape(B, NC * NCH * N, CW).reshape(B, NC, NCH * N, CW),
                  scales.reshape(B, N, L))
  (rp,) = _propagate(dst2, src2, ah3, oe.reshape(B * NCH * N, CW))
  out = _combine_n(rp.reshape(B, NC, NCH * N, CW), scales.reshape(B, N, L))
  return out
